# Initial kernel scaffold; baseline (speedup 1.0000x reference)
import jax
import jax.numpy as jnp
from jax.experimental import pallas as pl


def kernel(heatmap_logits):
    raise NotImplementedError("write your pallas kernel here")


# speedup vs baseline: 7.4894x; 7.4894x over previous
import functools

import jax
import jax.numpy as jnp
from jax import lax
from jax.experimental import pallas as pl
from jax.experimental.pallas import tpu as pltpu

_TOPK = 100
_THRESH = -1000000000.0
_NEG = -1000000000.0


def _halve_max(cur, w):
    while w > 1:
        w //= 2
        cur = jnp.maximum(cur[:, :w, :], cur[:, w:, :])
    return cur


def _make_body(R, W, H, S1, N1, G, N0, topk):
    def body(x_ref, score3_ref, view3_ref, row3_ref, col3_ref, m_ref, v1_ref, v0_ref):
        score_ref = score3_ref.at[0]
        view_ref = view3_ref.at[0]
        row_ref = row3_ref.at[0]
        col_ref = col3_ref.at[0]
        x = x_ref[0]
        ninf = jnp.float32(-jnp.inf)
        ncol1 = jnp.full((R, 1), ninf, jnp.float32)
        ncol2 = jnp.full((R, 2), ninf, jnp.float32)
        nrow1 = jnp.full((1, W), ninf, jnp.float32)
        nrow2 = jnp.full((2, W), ninf, jnp.float32)
        h = jnp.maximum(
            jnp.maximum(x, jnp.concatenate([x[:, 1:], ncol1], 1)),
            jnp.concatenate([ncol1, x[:, :-1]], 1),
        )
        h = jnp.maximum(
            h,
            jnp.maximum(
                jnp.concatenate([x[:, 2:], ncol2], 1),
                jnp.concatenate([ncol2, x[:, :-2]], 1),
            ),
        )
        vv = jnp.maximum(
            jnp.maximum(h, jnp.concatenate([h[1:, :], nrow1], 0)),
            jnp.concatenate([nrow1, h[:-1, :]], 0),
        )
        vv = jnp.maximum(
            vv,
            jnp.maximum(
                jnp.concatenate([h[2:, :], nrow2], 0),
                jnp.concatenate([nrow2, h[:-2, :]], 0),
            ),
        )
        m = jnp.where(x == vv, x, jnp.float32(_NEG))
        m_ref[...] = m

        v1 = _halve_max(m.reshape(N1, S1, W), S1).reshape(N1, W)
        v1_ref[...] = v1
        v0_ref[...] = _halve_max(v1.reshape(N0, G, W), G).reshape(N0, W)

        score_ref[...] = jnp.zeros((1, 128), jnp.float32)
        view_ref[...] = jnp.zeros((1, 128), jnp.float32)
        row_ref[...] = jnp.zeros((1, 128), jnp.float32)
        col_ref[...] = jnp.zeros((1, 128), jnp.float32)

        lane128 = lax.broadcasted_iota(jnp.int32, (1, 128), 1)
        iota0r = lax.broadcasted_iota(jnp.int32, (N0, W), 0)
        iotagr = lax.broadcasted_iota(jnp.int32, (G, W), 0)
        iotasr = lax.broadcasted_iota(jnp.int32, (S1, W), 0)
        iotac = lax.broadcasted_iota(jnp.int32, (1, W), 1)

        def iter_body(i, carry):
            v0 = v0_ref[...]
            vmax = jnp.max(v0)
            s0 = jnp.min(jnp.where(v0 == vmax, iota0r, N0))
            v1g = v1_ref[pl.ds(s0 * G, G), :]
            s1 = s0 * G + jnp.min(jnp.where(v1g == vmax, iotagr, G))
            ms = m_ref[pl.ds(s1 * S1, S1), :]
            r = s1 * S1 + jnp.min(jnp.where(ms == vmax, iotasr, S1))
            rowv = m_ref[pl.ds(r, 1), :]
            c = jnp.min(jnp.where(rowv == vmax, iotac, W))

            lm = lane128 == i
            score_ref[...] = jnp.where(lm, vmax, score_ref[...])
            view_ref[...] = jnp.where(lm, (r // H).astype(jnp.float32), view_ref[...])
            row_ref[...] = jnp.where(lm, (r % H).astype(jnp.float32), row_ref[...])
            col_ref[...] = jnp.where(lm, c.astype(jnp.float32), col_ref[...])

            m_ref[pl.ds(r, 1), :] = jnp.where(iotac == c, ninf, rowv)
            v1row = jnp.max(m_ref[pl.ds(s1 * S1, S1), :], axis=0, keepdims=True)
            v1_ref[pl.ds(s1, 1), :] = v1row
            v0row = jnp.max(v1_ref[pl.ds(s0 * G, G), :], axis=0, keepdims=True)
            v0_ref[pl.ds(s0, 1), :] = v0row
            return carry

        lax.fori_loop(0, topk, iter_body, 0)

    return body


@functools.partial(jax.jit, static_argnums=())
def kernel(heatmap_logits):
    bs, num_img, _, H, W = heatmap_logits.shape
    R = num_img * H
    hm = heatmap_logits.reshape(bs, R, W)

    S1 = 32
    N1 = R // S1
    G = 16 if N1 % 16 == 0 else N1
    N0 = N1 // G
    topk = min(_TOPK, R * W)

    body = _make_body(R, W, H, S1, N1, G, N0, topk)
    outs = pl.pallas_call(
        body,
        grid=(bs,),
        in_specs=[pl.BlockSpec((1, R, W), lambda b: (b, 0, 0))],
        out_specs=[pl.BlockSpec((1, 1, 128), lambda b: (b, 0, 0)) for _ in range(4)],
        out_shape=[jax.ShapeDtypeStruct((bs, 1, 128), jnp.float32) for _ in range(4)],
        scratch_shapes=[
            pltpu.VMEM((R, W), jnp.float32),
            pltpu.VMEM((N1, W), jnp.float32),
            pltpu.VMEM((N0, W), jnp.float32),
        ],
    )(hm)
    scores128, views128, rows128, cols128 = [o[:, 0, :] for o in outs]
    scores = scores128[:, :topk]
    peak_positions = jnp.stack(
        [views128[:, :topk], rows128[:, :topk], cols128[:, :topk]], axis=-1
    )
    peak_mask = scores > _THRESH
    return peak_positions, scores, peak_mask
